# baseline (speedup 1.0000x reference)
import jax
import jax.numpy as jnp
from jax import lax
from jax.experimental import pallas as pl
from jax.experimental.pallas import tpu as pltpu
from jax.experimental.pallas import tpu_sc as plsc

EDIM = 64
BATCH = 16384

_NC = 2
_NS = 16
_NW = _NC * _NS
_BPW = BATCH // _NW
_G = 16
_NG = _BPW // _G


def _sc_gather_body(x2_hbm, x3_hbm, e2_hbm, e3_hbm, out2_hbm, out3_hbm,
                    idx_s, sem0, sem1, sem2, sem3):
  sems = (sem0, sem1, sem2, sem3)
  wid = lax.axis_index("s") * _NC + lax.axis_index("c")
  base = wid * _BPW
  pltpu.sync_copy(x2_hbm.at[pl.ds(base, _BPW)], idx_s.at[0])
  pltpu.sync_copy(x3_hbm.at[pl.ds(base, _BPW)], idx_s.at[1])
  for t, (e_hbm, out_hbm) in enumerate(((e2_hbm, out2_hbm),
                                        (e3_hbm, out3_hbm))):
    def drain(grp, e_hbm=e_hbm, out_hbm=out_hbm):
      q = _G // 4
      for s in range(4):
        pltpu.make_async_copy(
            e_hbm.at[pl.ds(0, q)],
            out_hbm.at[pl.ds(base + grp * _G + s * q, q)], sems[s]).wait()

    def body(grp, carry, t=t, e_hbm=e_hbm, out_hbm=out_hbm, drain=drain):
      vec = idx_s[t, pl.ds(grp * _G, _G)]
      for k in range(_G):
        i = vec[k]
        pltpu.async_copy(e_hbm.at[pl.ds(i, 1)],
                         out_hbm.at[pl.ds(base + grp * _G + k, 1)],
                         sems[(k * 4) // _G])
      @pl.when(grp > 0)
      def _():
        drain(grp - 1)
      return carry

    lax.fori_loop(0, _NG, body, 0)
    drain(_NG - 1)


def _sc_gather(x2, x3, e2, e3):
  mesh = plsc.VectorSubcoreMesh(core_axis_name="c", subcore_axis_name="s")
  f = pl.kernel(
      _sc_gather_body,
      mesh=mesh,
      out_type=(
          jax.ShapeDtypeStruct((BATCH, EDIM), jnp.float32),
          jax.ShapeDtypeStruct((BATCH, EDIM), jnp.float32),
      ),
      scratch_types=[
          pltpu.VMEM((2, _BPW), jnp.int32),
          pltpu.SemaphoreType.DMA,
          pltpu.SemaphoreType.DMA,
          pltpu.SemaphoreType.DMA,
          pltpu.SemaphoreType.DMA,
      ],
  )
  return f(x2, x3, e2, e3)


_BM = 2048


def _mlp_body(x2_ref, x3_ref, w1a_ref, w1b_ref, b1_ref, w2_ref, b2_ref,
              w3_ref, b3_ref, w4_ref, b4_ref, out_ref):
  h = jnp.dot(x2_ref[...], w1a_ref[...], preferred_element_type=jnp.float32)
  h = h + jnp.dot(x3_ref[...], w1b_ref[...], preferred_element_type=jnp.float32)
  h = jnp.maximum(h + b1_ref[...], 0.0)
  h = jnp.maximum(
      jnp.dot(h, w2_ref[...], preferred_element_type=jnp.float32) + b2_ref[...],
      0.0)
  h = jnp.maximum(
      jnp.dot(h, w3_ref[...], preferred_element_type=jnp.float32) + b3_ref[...],
      0.0)
  out_ref[...] = (
      jnp.dot(h, w4_ref[...], preferred_element_type=jnp.float32) + b4_ref[...])


def _mlp(emb2, emb3, W1, b1, W2, b2, W3, b3, W4, b4):
  w1a = W1[:EDIM]
  w1b = W1[EDIM:]
  full = lambda i: (0, 0)
  return pl.pallas_call(
      _mlp_body,
      grid=(BATCH // _BM,),
      in_specs=[
          pl.BlockSpec((_BM, EDIM), lambda i: (i, 0)),
          pl.BlockSpec((_BM, EDIM), lambda i: (i, 0)),
          pl.BlockSpec(w1a.shape, full),
          pl.BlockSpec(w1b.shape, full),
          pl.BlockSpec((1, 32), full),
          pl.BlockSpec(W2.shape, full),
          pl.BlockSpec((1, 32), full),
          pl.BlockSpec(W3.shape, full),
          pl.BlockSpec((1, 16), full),
          pl.BlockSpec(W4.shape, full),
          pl.BlockSpec((1, 3), full),
      ],
      out_specs=pl.BlockSpec((_BM, 3), lambda i: (i, 0)),
      out_shape=jax.ShapeDtypeStruct((BATCH, 3), jnp.float32),
  )(emb2, emb3, w1a, w1b, b1.reshape(1, 32), W2, b2.reshape(1, 32),
    W3, b3.reshape(1, 16), W4, b4.reshape(1, 3))


def kernel(X_2, X_3, E2, E3, W1, b1, W2, b2, W3, b3, W4, b4):
  emb2, emb3 = _sc_gather(X_2.astype(jnp.int32), X_3.astype(jnp.int32), E2, E3)
  return _mlp(emb2, emb3, W1, b1, W2, b2, W3, b3, W4, b4)


# speedup vs baseline: 2.0781x; 2.0781x over previous
import jax
import jax.numpy as jnp
from jax import lax
from jax.experimental import pallas as pl
from jax.experimental.pallas import tpu as pltpu
from jax.experimental.pallas import tpu_sc as plsc

EDIM = 64
BATCH = 16384

_NC = 2
_NS = 16
_NW = _NC * _NS
_BPW = BATCH // _NW
_G = 16
_NG = _BPW // _G


def _sc_gather_body(x2_hbm, x3_hbm, e2_hbm, e3_hbm, out2_hbm, out3_hbm,
                    idx_s, rows_v, sem0, sem1, sem2, sem3):
  sems = (sem0, sem1, sem2, sem3)
  wid = lax.axis_index("s") * _NC + lax.axis_index("c")
  base = wid * _BPW
  pltpu.sync_copy(x2_hbm.at[pl.ds(base, _BPW)], idx_s.at[0])
  pltpu.sync_copy(x3_hbm.at[pl.ds(base, _BPW)], idx_s.at[1])
  for t, (e_hbm, out_hbm) in enumerate(((e2_hbm, out2_hbm),
                                        (e3_hbm, out3_hbm))):
    def drain(grp, e_hbm=e_hbm):
      q = _G // 4
      for s in range(4):
        pltpu.make_async_copy(
            e_hbm.at[pl.ds(0, q)],
            rows_v.at[pl.ds(grp * _G + s * q, q)], sems[s]).wait()

    def body(grp, carry, t=t, e_hbm=e_hbm, drain=drain):
      vec = idx_s[t, pl.ds(grp * _G, _G)]
      for k in range(_G):
        i = vec[k]
        pltpu.async_copy(e_hbm.at[pl.ds(i, 1)],
                         rows_v.at[pl.ds(grp * _G + k, 1)],
                         sems[(k * 4) // _G])
      @pl.when(grp > 0)
      def _():
        drain(grp - 1)
      return carry

    lax.fori_loop(0, _NG, body, 0)
    drain(_NG - 1)
    pltpu.sync_copy(rows_v, out_hbm.at[pl.ds(base, _BPW)])


def _sc_gather(x2, x3, e2, e3):
  mesh = plsc.VectorSubcoreMesh(core_axis_name="c", subcore_axis_name="s")
  f = pl.kernel(
      _sc_gather_body,
      mesh=mesh,
      out_type=(
          jax.ShapeDtypeStruct((BATCH, EDIM), jnp.float32),
          jax.ShapeDtypeStruct((BATCH, EDIM), jnp.float32),
      ),
      scratch_types=[
          pltpu.VMEM((2, _BPW), jnp.int32),
          pltpu.VMEM((_BPW, EDIM), jnp.float32),
          pltpu.SemaphoreType.DMA,
          pltpu.SemaphoreType.DMA,
          pltpu.SemaphoreType.DMA,
          pltpu.SemaphoreType.DMA,
      ],
  )
  return f(x2, x3, e2, e3)


_BM = 2048


def _mlp_body(x2_ref, x3_ref, w1a_ref, w1b_ref, b1_ref, w2_ref, b2_ref,
              w3_ref, b3_ref, w4_ref, b4_ref, out_ref):
  h = jnp.dot(x2_ref[...], w1a_ref[...], preferred_element_type=jnp.float32)
  h = h + jnp.dot(x3_ref[...], w1b_ref[...], preferred_element_type=jnp.float32)
  h = jnp.maximum(h + b1_ref[...], 0.0)
  h = jnp.maximum(
      jnp.dot(h, w2_ref[...], preferred_element_type=jnp.float32) + b2_ref[...],
      0.0)
  h = jnp.maximum(
      jnp.dot(h, w3_ref[...], preferred_element_type=jnp.float32) + b3_ref[...],
      0.0)
  out_ref[...] = (
      jnp.dot(h, w4_ref[...], preferred_element_type=jnp.float32) + b4_ref[...])


def _mlp(emb2, emb3, W1, b1, W2, b2, W3, b3, W4, b4):
  w1a = W1[:EDIM]
  w1b = W1[EDIM:]
  full = lambda i: (0, 0)
  return pl.pallas_call(
      _mlp_body,
      grid=(BATCH // _BM,),
      in_specs=[
          pl.BlockSpec((_BM, EDIM), lambda i: (i, 0)),
          pl.BlockSpec((_BM, EDIM), lambda i: (i, 0)),
          pl.BlockSpec(w1a.shape, full),
          pl.BlockSpec(w1b.shape, full),
          pl.BlockSpec((1, 32), full),
          pl.BlockSpec(W2.shape, full),
          pl.BlockSpec((1, 32), full),
          pl.BlockSpec(W3.shape, full),
          pl.BlockSpec((1, 16), full),
          pl.BlockSpec(W4.shape, full),
          pl.BlockSpec((1, 3), full),
      ],
      out_specs=pl.BlockSpec((_BM, 3), lambda i: (i, 0)),
      out_shape=jax.ShapeDtypeStruct((BATCH, 3), jnp.float32),
  )(emb2, emb3, w1a, w1b, b1.reshape(1, 32), W2, b2.reshape(1, 32),
    W3, b3.reshape(1, 16), W4, b4.reshape(1, 3))


def kernel(X_2, X_3, E2, E3, W1, b1, W2, b2, W3, b3, W4, b4):
  emb2, emb3 = _sc_gather(X_2.astype(jnp.int32), X_3.astype(jnp.int32), E2, E3)
  return _mlp(emb2, emb3, W1, b1, W2, b2, W3, b3, W4, b4)
